# baseline (speedup 1.0000x reference)
import functools

import jax
import jax.numpy as jnp
from jax import lax
from jax.experimental import pallas as pl
from jax.experimental.pallas import tpu as pltpu
from jax.experimental.pallas import tpu_sc as plsc

N_NODES = 10000
N_PAD = 10240
CH = 256
HALF = 128
NC = 2
NS = 16
L = 16
CHUNK = 128
CPT = 80
NSEG = 5
CPS = CPT // NSEG
NBUF = 2
EDGES_PER_TILE = CHUNK * CPT
E_PAD = NS * EDGES_PER_TILE
EPW = E_PAD // (NC * NS)
ROWS_PER_TILE = N_PAD // NS
ROW_BLOCK = 512
GRID = N_PAD // ROW_BLOCK

_sc_mesh = plsc.VectorSubcoreMesh(
    core_axis_name="c", subcore_axis_name="s", num_cores=NC, num_subcores=NS)


@functools.partial(
    pl.kernel,
    out_type=jax.ShapeDtypeStruct((NC * NS, N_PAD), jnp.float32),
    mesh=_sc_mesh,
    scratch_types=[
        pltpu.VMEM((EPW,), jnp.int32),
        pltpu.VMEM((N_PAD,), jnp.float32),
    ],
    compiler_params=pltpu.CompilerParams(needs_layout_passes=False),
)
def _deg_kernel(dst_hbm, out_hbm, idx_v, deg_v):
    cid = lax.axis_index("c")
    sid = lax.axis_index("s")
    wid = cid * NS + sid
    zeros = jnp.zeros((L,), jnp.float32)

    def zero_body(i, _):
        for u in range(8):
            deg_v[pl.ds(i * 8 * L + u * L, L)] = zeros
        return 0

    lax.fori_loop(0, N_PAD // (8 * L), zero_body, 0)
    pltpu.sync_copy(dst_hbm.at[pl.ds(wid * EPW, EPW)], idx_v)
    ones = jnp.ones((L,), jnp.float32)

    def body(i, _):
        for u in range(4):
            idx = idx_v[pl.ds(i * 4 * L + u * L, L)]
            plsc.addupdate_scatter(deg_v, [idx], ones)
        return 0

    lax.fori_loop(0, EPW // (4 * L), body, 0)
    pltpu.sync_copy(deg_v, out_hbm.at[wid])


def _mm_body(x_ref, w_ref, degp_ref, h2a_ref, h2b_ref, dinv_ref):
    h = jnp.dot(x_ref[...], w_ref[...], preferred_element_type=jnp.float32)
    deg = jnp.sum(degp_ref[...], axis=0, keepdims=True) + 1.0
    dinv = jnp.transpose(lax.rsqrt(deg))
    h2 = h * dinv
    h2a_ref[...] = h2[:, :HALF]
    h2b_ref[...] = h2[:, HALF:]
    dinv_ref[...] = dinv


def _mm_call(x, W, degp):
    return pl.pallas_call(
        _mm_body,
        grid=(GRID,),
        in_specs=[
            pl.BlockSpec((ROW_BLOCK, CH), lambda i: (i, 0)),
            pl.BlockSpec((CH, CH), lambda i: (0, 0)),
            pl.BlockSpec((NC * NS, ROW_BLOCK), lambda i: (0, i)),
        ],
        out_specs=[
            pl.BlockSpec((ROW_BLOCK, HALF), lambda i: (i, 0)),
            pl.BlockSpec((ROW_BLOCK, HALF), lambda i: (i, 0)),
            pl.BlockSpec((ROW_BLOCK, 1), lambda i: (i, 0)),
        ],
        out_shape=[
            jax.ShapeDtypeStruct((N_PAD, HALF), jnp.float32),
            jax.ShapeDtypeStruct((N_PAD, HALF), jnp.float32),
            jax.ShapeDtypeStruct((N_PAD, 1), jnp.float32),
        ],
    )(x, W, degp)


@functools.partial(
    pl.kernel,
    out_type=[
        jax.ShapeDtypeStruct((N_PAD, HALF), jnp.float32),
        jax.ShapeDtypeStruct((N_PAD, HALF), jnp.float32),
    ],
    mesh=_sc_mesh,
    scratch_types=[
        pltpu.VMEM((CPS, CHUNK), jnp.int32),
        pltpu.VMEM((CPS, CHUNK), jnp.int32),
        [pltpu.VMEM((CHUNK, HALF), jnp.float32) for _ in range(NBUF)],
        pltpu.VMEM_SHARED((N_PAD, HALF), jnp.float32),
        [pltpu.SemaphoreType.DMA for _ in range(NBUF)],
    ],
)
def _agg_kernel(h2a_hbm, h2b_hbm, src_hbm, dst_hbm, outa_hbm, outb_hbm,
                src_v, dst_v, bufs, acc_sh, gsems):
    cid = lax.axis_index("c")
    sid = lax.axis_index("s")
    row0 = sid * ROWS_PER_TILE
    base = sid * CPT

    def run_half(table_hbm, out_hbm):
        pltpu.sync_copy(table_hbm.at[pl.ds(row0, ROWS_PER_TILE)],
                        acc_sh.at[pl.ds(row0, ROWS_PER_TILE)])
        plsc.subcore_barrier()

        def seg_body(seg, _):
            segbase = base + seg * CPS
            pltpu.sync_copy(src_hbm.at[pl.ds(segbase, CPS)], src_v)
            pltpu.sync_copy(dst_hbm.at[pl.ds(segbase, CPS)], dst_v)

            pltpu.async_copy(table_hbm.at[src_v.at[0]], bufs[0], gsems[0])

            def pair_body(p, _):
                j0 = 2 * p
                pltpu.async_copy(table_hbm.at[src_v.at[j0 + 1]], bufs[1],
                                 gsems[1])
                pltpu.make_async_copy(
                    table_hbm.at[src_v.at[j0]], bufs[0], gsems[0]).wait()
                pltpu.sync_copy(bufs[0], acc_sh.at[dst_v.at[j0]], add=True)

                @pl.when(p < CPS // 2 - 1)
                def _():
                    pltpu.async_copy(table_hbm.at[src_v.at[j0 + 2]], bufs[0],
                                     gsems[0])

                pltpu.make_async_copy(
                    table_hbm.at[src_v.at[j0 + 1]], bufs[1], gsems[1]).wait()
                pltpu.sync_copy(bufs[1], acc_sh.at[dst_v.at[j0 + 1]], add=True)
                return 0

            lax.fori_loop(0, CPS // 2, pair_body, 0)
            return 0

        lax.fori_loop(0, NSEG, seg_body, 0)
        plsc.subcore_barrier()
        pltpu.sync_copy(acc_sh.at[pl.ds(row0, ROWS_PER_TILE)],
                        out_hbm.at[pl.ds(row0, ROWS_PER_TILE)])

    @pl.when(cid == 0)
    def _():
        run_half(h2a_hbm, outa_hbm)

    @pl.when(cid == 1)
    def _():
        run_half(h2b_hbm, outb_hbm)


def _final_body(acca_ref, accb_ref, dinv_ref, b_ref, out_ref):
    dinv = dinv_ref[...]
    br = b_ref[...]
    out_ref[:, :HALF] = dinv * acca_ref[...] + br[0:1, :HALF]
    out_ref[:, HALF:] = dinv * accb_ref[...] + br[0:1, HALF:]


def _final_call(acca, accb, dinv, b2d):
    return pl.pallas_call(
        _final_body,
        grid=(GRID,),
        in_specs=[
            pl.BlockSpec((ROW_BLOCK, HALF), lambda i: (i, 0)),
            pl.BlockSpec((ROW_BLOCK, HALF), lambda i: (i, 0)),
            pl.BlockSpec((ROW_BLOCK, 1), lambda i: (i, 0)),
            pl.BlockSpec((1, CH), lambda i: (0, 0)),
        ],
        out_specs=pl.BlockSpec((ROW_BLOCK, CH), lambda i: (i, 0)),
        out_shape=jax.ShapeDtypeStruct((N_NODES, CH), jnp.float32),
    )(acca, accb, dinv, b2d)


def kernel(x, edge_index, W, b):
    e = edge_index.shape[1]
    src = edge_index[0].astype(jnp.int32)
    dst = edge_index[1].astype(jnp.int32)
    pad = E_PAD - e
    src_p = jnp.concatenate([src, jnp.full((pad,), N_PAD - 1, jnp.int32)])
    dst_p = jnp.concatenate([dst, jnp.full((pad,), N_PAD - 1, jnp.int32)])
    src2d = src_p.reshape(NS * CPT, CHUNK)
    dst2d = dst_p.reshape(NS * CPT, CHUNK)

    degp = _deg_kernel(dst_p)
    h2a, h2b, dinv = _mm_call(x, W, degp)
    acca, accb = _agg_kernel(h2a, h2b, src2d, dst2d)
    return _final_call(acca, accb, dinv, b.reshape(1, CH))


# speedup vs baseline: 1.0236x; 1.0236x over previous
import functools

import jax
import jax.numpy as jnp
from jax import lax
from jax.experimental import pallas as pl
from jax.experimental.pallas import tpu as pltpu
from jax.experimental.pallas import tpu_sc as plsc

N_NODES = 10000
N_PAD = 10240
CH = 256
HALF = 128
NC = 2
NS = 16
L = 16
CHUNK = 128
CPT = 80
NSEG = 5
CPS = CPT // NSEG
NBUF = 2
EDGES_PER_TILE = CHUNK * CPT
E_PAD = NS * EDGES_PER_TILE
EPW = E_PAD // (NC * NS)
ROWS_PER_TILE = N_PAD // NS
ROW_BLOCK = 512
GRID = N_PAD // ROW_BLOCK

_sc_mesh = plsc.VectorSubcoreMesh(
    core_axis_name="c", subcore_axis_name="s", num_cores=NC, num_subcores=NS)


@functools.partial(
    pl.kernel,
    out_type=jax.ShapeDtypeStruct((NC * NS, N_PAD), jnp.float32),
    mesh=_sc_mesh,
    scratch_types=[
        pltpu.VMEM((EPW,), jnp.int32),
        pltpu.VMEM((N_PAD,), jnp.float32),
    ],
    compiler_params=pltpu.CompilerParams(needs_layout_passes=False),
)
def _deg_kernel(dst_hbm, out_hbm, idx_v, deg_v):
    cid = lax.axis_index("c")
    sid = lax.axis_index("s")
    wid = cid * NS + sid
    zeros = jnp.zeros((L,), jnp.float32)

    def zero_body(i, _):
        for u in range(8):
            deg_v[pl.ds(i * 8 * L + u * L, L)] = zeros
        return 0

    lax.fori_loop(0, N_PAD // (8 * L), zero_body, 0)
    pltpu.sync_copy(dst_hbm.at[pl.ds(wid * EPW, EPW)], idx_v)
    ones = jnp.ones((L,), jnp.float32)

    def body(i, _):
        for u in range(4):
            idx = idx_v[pl.ds(i * 4 * L + u * L, L)]
            plsc.addupdate_scatter(deg_v, [idx], ones)
        return 0

    lax.fori_loop(0, EPW // (4 * L), body, 0)
    pltpu.sync_copy(deg_v, out_hbm.at[wid])


def _mm_body(x_ref, w_ref, degp_ref, h2a_ref, h2b_ref, dinv_ref):
    h = jnp.dot(x_ref[...].astype(jnp.bfloat16),
                w_ref[...].astype(jnp.bfloat16),
                preferred_element_type=jnp.float32)
    deg = jnp.sum(degp_ref[...], axis=1, keepdims=True) + 1.0
    dinv = lax.rsqrt(deg)
    h2 = h * dinv
    h2a_ref[...] = h2[:, :HALF]
    h2b_ref[...] = h2[:, HALF:]
    dinv_ref[...] = dinv


def _mm_call(x_pad, W, degp_t):
    return pl.pallas_call(
        _mm_body,
        grid=(GRID,),
        in_specs=[
            pl.BlockSpec((ROW_BLOCK, CH), lambda i: (i, 0)),
            pl.BlockSpec((CH, CH), lambda i: (0, 0)),
            pl.BlockSpec((ROW_BLOCK, NC * NS), lambda i: (i, 0)),
        ],
        out_specs=[
            pl.BlockSpec((ROW_BLOCK, HALF), lambda i: (i, 0)),
            pl.BlockSpec((ROW_BLOCK, HALF), lambda i: (i, 0)),
            pl.BlockSpec((ROW_BLOCK, 1), lambda i: (i, 0)),
        ],
        out_shape=[
            jax.ShapeDtypeStruct((N_PAD, HALF), jnp.float32),
            jax.ShapeDtypeStruct((N_PAD, HALF), jnp.float32),
            jax.ShapeDtypeStruct((N_PAD, 1), jnp.float32),
        ],
    )(x_pad, W, degp_t)


@functools.partial(
    pl.kernel,
    out_type=[
        jax.ShapeDtypeStruct((N_PAD, HALF), jnp.float32),
        jax.ShapeDtypeStruct((N_PAD, HALF), jnp.float32),
    ],
    mesh=_sc_mesh,
    scratch_types=[
        pltpu.VMEM((CPS, CHUNK), jnp.int32),
        pltpu.VMEM((CPS, CHUNK), jnp.int32),
        [pltpu.VMEM((CHUNK, HALF), jnp.float32) for _ in range(NBUF)],
        pltpu.VMEM_SHARED((N_PAD, HALF), jnp.float32),
        [pltpu.SemaphoreType.DMA for _ in range(NBUF)],
    ],
)
def _agg_kernel(h2a_hbm, h2b_hbm, src_hbm, dst_hbm, outa_hbm, outb_hbm,
                src_v, dst_v, bufs, acc_sh, gsems):
    cid = lax.axis_index("c")
    sid = lax.axis_index("s")
    row0 = sid * ROWS_PER_TILE
    base = sid * CPT

    def run_half(table_hbm, out_hbm):
        pltpu.sync_copy(table_hbm.at[pl.ds(row0, ROWS_PER_TILE)],
                        acc_sh.at[pl.ds(row0, ROWS_PER_TILE)])
        plsc.subcore_barrier()

        def seg_body(seg, _):
            segbase = base + seg * CPS
            pltpu.sync_copy(src_hbm.at[pl.ds(segbase, CPS)], src_v)
            pltpu.sync_copy(dst_hbm.at[pl.ds(segbase, CPS)], dst_v)

            pltpu.async_copy(table_hbm.at[src_v.at[0]], bufs[0], gsems[0])

            def pair_body(p, _):
                j0 = 2 * p
                pltpu.async_copy(table_hbm.at[src_v.at[j0 + 1]], bufs[1],
                                 gsems[1])
                pltpu.make_async_copy(
                    table_hbm.at[src_v.at[j0]], bufs[0], gsems[0]).wait()
                pltpu.sync_copy(bufs[0], acc_sh.at[dst_v.at[j0]], add=True)

                @pl.when(p < CPS // 2 - 1)
                def _():
                    pltpu.async_copy(table_hbm.at[src_v.at[j0 + 2]], bufs[0],
                                     gsems[0])

                pltpu.make_async_copy(
                    table_hbm.at[src_v.at[j0 + 1]], bufs[1], gsems[1]).wait()
                pltpu.sync_copy(bufs[1], acc_sh.at[dst_v.at[j0 + 1]], add=True)
                return 0

            lax.fori_loop(0, CPS // 2, pair_body, 0)
            return 0

        lax.fori_loop(0, NSEG, seg_body, 0)
        plsc.subcore_barrier()
        pltpu.sync_copy(acc_sh.at[pl.ds(row0, ROWS_PER_TILE)],
                        out_hbm.at[pl.ds(row0, ROWS_PER_TILE)])

    @pl.when(cid == 0)
    def _():
        run_half(h2a_hbm, outa_hbm)

    @pl.when(cid == 1)
    def _():
        run_half(h2b_hbm, outb_hbm)


def _final_body(acca_ref, accb_ref, dinv_ref, b_ref, out_ref):
    dinv = dinv_ref[...]
    br = b_ref[...]
    out_ref[:, :HALF] = dinv * acca_ref[...] + br[0:1, :HALF]
    out_ref[:, HALF:] = dinv * accb_ref[...] + br[0:1, HALF:]


def _final_call(acca, accb, dinv, b2d):
    return pl.pallas_call(
        _final_body,
        grid=(GRID,),
        in_specs=[
            pl.BlockSpec((ROW_BLOCK, HALF), lambda i: (i, 0)),
            pl.BlockSpec((ROW_BLOCK, HALF), lambda i: (i, 0)),
            pl.BlockSpec((ROW_BLOCK, 1), lambda i: (i, 0)),
            pl.BlockSpec((1, CH), lambda i: (0, 0)),
        ],
        out_specs=pl.BlockSpec((ROW_BLOCK, CH), lambda i: (i, 0)),
        out_shape=jax.ShapeDtypeStruct((N_PAD, CH), jnp.float32),
    )(acca, accb, dinv, b2d)


def kernel(x, edge_index, W, b):
    n, _ = x.shape
    e = edge_index.shape[1]
    src = edge_index[0].astype(jnp.int32)
    dst = edge_index[1].astype(jnp.int32)
    pad = E_PAD - e
    src_p = jnp.concatenate([src, jnp.full((pad,), n, jnp.int32)])
    dst_p = jnp.concatenate([dst, jnp.full((pad,), n, jnp.int32)])
    src2d = src_p.reshape(NS * CPT, CHUNK)
    dst2d = dst_p.reshape(NS * CPT, CHUNK)
    x_pad = jnp.pad(x, ((0, N_PAD - n), (0, 0)))

    degp = _deg_kernel(dst_p)
    degp_t = degp.T
    h2a, h2b, dinv = _mm_call(x_pad, W, degp_t)
    acca, accb = _agg_kernel(h2a, h2b, src2d, dst2d)
    out = _final_call(acca, accb, dinv, b.reshape(1, CH))
    return out[:n]
